# Initial kernel scaffold; baseline (speedup 1.0000x reference)
import jax
import jax.numpy as jnp
from jax.experimental import pallas as pl


def kernel(x, edge_index, W, b):
    raise NotImplementedError("write your pallas kernel here")


# speedup vs baseline: 17.6251x; 17.6251x over previous
import functools

import jax
import jax.numpy as jnp
from jax import lax
from jax.experimental import pallas as pl
from jax.experimental.pallas import tpu as pltpu, tpu_sc as plsc

NC = 2
NS = 16
NW = NC * NS
L = 16
CB = 128

F32 = jnp.float32
I32 = jnp.int32


def _sc_mesh():
    return plsc.VectorSubcoreMesh(
        core_axis_name="c", subcore_axis_name="s", num_cores=NC, num_subcores=NS
    )


def _zero_vmem_rows(ref, nrows, ncols):
    z = jnp.zeros((L,), F32)

    def body(i, _):
        for g in range(ncols // L):
            ref[i, pl.ds(g * L, L)] = z
        return 0

    lax.fori_loop(0, nrows, body, 0)


def _sc_degree(dst_p, n_pad, chunks):
    tpe = chunks * CB
    tn = n_pad // NS

    @functools.partial(
        pl.kernel,
        out_type=jax.ShapeDtypeStruct((NC, n_pad), F32),
        mesh=_sc_mesh(),
        scratch_types=[
            pltpu.VMEM((CB,), I32),
            pltpu.VMEM((CB,), F32),
            pltpu.VMEM((tn,), F32),
            pltpu.VMEM_SHARED((n_pad,), F32),
        ],
    )
    def k(dst_hbm, deg_hbm, idx_v, ones_v, slice_v, deg_sh):
        c = lax.axis_index("c")
        s = lax.axis_index("s")
        one = jnp.ones((L,), F32)
        zero = jnp.zeros((L,), F32)
        for g in range(CB // L):
            ones_v[pl.ds(g * L, L)] = one

        def zbody(i, _):
            slice_v[pl.ds(i * L, L)] = zero
            return 0

        lax.fori_loop(0, tn // L, zbody, 0)
        pltpu.sync_copy(slice_v, deg_sh.at[pl.ds(s * tn, tn)])
        plsc.subcore_barrier()

        base = (c * NS + s) * tpe

        def ebody(g, _):
            pltpu.sync_copy(dst_hbm.at[pl.ds(base + g * CB, CB)], idx_v)
            pltpu.sync_copy(ones_v, deg_sh.at[idx_v], add=True)
            return 0

        lax.fori_loop(0, chunks, ebody, 0)
        plsc.subcore_barrier()

        pltpu.sync_copy(deg_sh.at[pl.ds(s * tn, tn)], slice_v)
        pltpu.sync_copy(slice_v, deg_hbm.at[c, pl.ds(s * tn, tn)])

    return k(dst_p)


def _sc_aggregate(xwn, src_p, dst_p, n_pad, chunks):
    tpe = chunks * CB
    tn = n_pad // NS
    D = xwn.shape[1]

    @functools.partial(
        pl.kernel,
        out_type=jax.ShapeDtypeStruct((NC, n_pad, D), F32),
        mesh=_sc_mesh(),
        scratch_types=[
            pltpu.VMEM((CB,), I32),
            pltpu.VMEM((CB,), I32),
            pltpu.VMEM((CB,), I32),
            pltpu.VMEM((CB,), I32),
            pltpu.VMEM((CB, D), F32),
            pltpu.VMEM((CB, D), F32),
            pltpu.SemaphoreType.DMA,
            pltpu.SemaphoreType.DMA,
            pltpu.VMEM_SHARED((n_pad, D), F32),
        ],
    )
    def k(xwn_hbm, src_hbm, dst_hbm, s_hbm,
          sidx_a, sidx_b, didx_a, didx_b, rows_a, rows_b, sem_a, sem_b,
          agg_sh):
        c = lax.axis_index("c")
        s = lax.axis_index("s")

        _zero_vmem_rows(rows_a, CB, D)
        for kk in range(tn // CB):
            pltpu.sync_copy(rows_a, agg_sh.at[pl.ds(s * tn + kk * CB, CB)])
        plsc.subcore_barrier()

        base = (c * NS + s) * tpe
        npairs = chunks // 2

        def stage_gather(g, sidx, rows, sem):
            pltpu.sync_copy(src_hbm.at[pl.ds(base + g * CB, CB)], sidx)
            return pltpu.async_copy(xwn_hbm.at[sidx], rows, sem)

        def scatter(g, didx, rows):
            pltpu.sync_copy(dst_hbm.at[pl.ds(base + g * CB, CB)], didx)
            pltpu.sync_copy(rows, agg_sh.at[didx], add=True)

        def ebody(i, _):
            g0 = i * 2
            g1 = g0 + 1
            cp_a = stage_gather(g0, sidx_a, rows_a, sem_a)
            cp_b = stage_gather(g1, sidx_b, rows_b, sem_b)
            cp_a.wait()
            scatter(g0, didx_a, rows_a)
            cp_b.wait()
            scatter(g1, didx_b, rows_b)
            return 0

        lax.fori_loop(0, npairs, ebody, 0)
        if chunks % 2:
            g = chunks - 1
            cp = stage_gather(g, sidx_a, rows_a, sem_a)
            cp.wait()
            scatter(g, didx_a, rows_a)
        plsc.subcore_barrier()

        for kk in range(tn // CB):
            sl = pl.ds(s * tn + kk * CB, CB)
            pltpu.sync_copy(agg_sh.at[sl], rows_a)
            pltpu.sync_copy(rows_a, s_hbm.at[c, sl])

    return k(xwn, src_p, dst_p)


def _tc_prescale(x_p, W, degs3):
    n_pad, D = x_p.shape
    BR = 1024

    def body(x_ref, w_ref, d0_ref, d1_ref, xwn_ref, dinv_ref):
        deg = d0_ref[0] + d1_ref[0] + 1.0
        dinv = lax.rsqrt(deg)
        xw = jnp.dot(x_ref[...], w_ref[...], preferred_element_type=F32)
        xwn_ref[...] = xw * dinv
        dinv_ref[...] = dinv

    return pl.pallas_call(
        body,
        grid=(n_pad // BR,),
        in_specs=[
            pl.BlockSpec((BR, D), lambda i: (i, 0)),
            pl.BlockSpec((D, D), lambda i: (0, 0)),
            pl.BlockSpec((1, BR, 1), lambda i: (0, i, 0)),
            pl.BlockSpec((1, BR, 1), lambda i: (1, i, 0)),
        ],
        out_specs=[
            pl.BlockSpec((BR, D), lambda i: (i, 0)),
            pl.BlockSpec((BR, 1), lambda i: (i, 0)),
        ],
        out_shape=[
            jax.ShapeDtypeStruct((n_pad, D), F32),
            jax.ShapeDtypeStruct((n_pad, 1), F32),
        ],
    )(x_p, W, degs3, degs3)


def _tc_combine(s_all, xwn, dinv, b, n):
    D = xwn.shape[1]
    BR = 1000

    def body(s0_ref, s1_ref, xwn_ref, dinv_ref, b_ref, out_ref):
        acc = s0_ref[0] + s1_ref[0] + xwn_ref[...]
        out_ref[...] = jnp.maximum(acc * dinv_ref[...] + b_ref[...], 0.0)

    return pl.pallas_call(
        body,
        grid=(n // BR,),
        in_specs=[
            pl.BlockSpec((1, BR, D), lambda i: (0, i, 0)),
            pl.BlockSpec((1, BR, D), lambda i: (1, i, 0)),
            pl.BlockSpec((BR, D), lambda i: (i, 0)),
            pl.BlockSpec((BR, 1), lambda i: (i, 0)),
            pl.BlockSpec((D,), lambda i: (0,)),
        ],
        out_specs=pl.BlockSpec((BR, D), lambda i: (i, 0)),
        out_shape=jax.ShapeDtypeStruct((n, D), F32),
    )(s_all, s_all, xwn, dinv, b)


def kernel(x, edge_index, W, b):
    n, D = x.shape
    E = edge_index.shape[1]
    n_pad = ((n + 255) // 256) * 256
    chunks = -(-E // (NW * CB))
    e_pad = NW * CB * chunks

    pad = jnp.full((e_pad - E,), n, I32)
    src_p = jnp.concatenate([edge_index[0], pad])
    dst_p = jnp.concatenate([edge_index[1], pad])
    x_p = jnp.pad(x, ((0, n_pad - n), (0, 0)))

    degs = _sc_degree(dst_p, n_pad, chunks)
    xwn, dinv = _tc_prescale(x_p, W, degs.reshape(NC, n_pad, 1))
    s_all = _sc_aggregate(xwn, src_p, dst_p, n_pad, chunks)
    return _tc_combine(s_all, xwn, dinv, b, n)
